# baseline (speedup 1.0000x reference)
import functools

import jax
import jax.numpy as jnp
from jax import lax
from jax.experimental import pallas as pl
from jax.experimental.pallas import tpu as pltpu
from jax.experimental.pallas import tpu_sc as plsc

_G = 32
_D = 768
_HALF = _D // 2
_BATCH = 64
_BB = 2
_TAIL = _BB
_HEAD = _BATCH - _TAIL


def _sc_concat_body(row_hbm, col_hbm, rowcol_hbm):
    cid = lax.axis_index("c")

    @pl.when(cid == 0)
    def _():
        pltpu.sync_copy(row_hbm, rowcol_hbm.at[:, pl.ds(0, _HALF)])

    @pl.when(cid == 1)
    def _():
        pltpu.sync_copy(col_hbm, rowcol_hbm.at[:, pl.ds(_HALF, _HALF)])


def _tc_head_body(row_ref, col_ref, out_ref):
    r = row_ref[...]
    c = col_ref[...]
    re = jnp.broadcast_to(r[:, None, :], (_G, _G, _HALF))
    ce = jnp.broadcast_to(c[None, :, :], (_G, _G, _HALF))
    pos = jnp.concatenate([re, ce], axis=-1)
    out_ref[...] = jnp.broadcast_to(pos[None], (_BB, _G, _G, _D))


def _tc_tail_body(rowcol_ref, part_ref, out_ref):
    del part_ref
    rc = rowcol_ref[...]
    r = rc[:, :_HALF]
    c = rc[:, _HALF:]
    re = jnp.broadcast_to(r[:, None, :], (_G, _G, _HALF))
    ce = jnp.broadcast_to(c[None, :, :], (_G, _G, _HALF))
    pos = jnp.concatenate([re, ce], axis=-1)
    out_ref[...] = pos[None]


def kernel(batch_size, row_table, col_table):
    del batch_size
    mesh = plsc.ScalarSubcoreMesh(axis_name="c", num_cores=2)
    sc_concat = functools.partial(
        pl.kernel,
        mesh=mesh,
        out_type=jax.ShapeDtypeStruct((_G, _D), jnp.float32),
    )(_sc_concat_body)
    part = pl.pallas_call(
        _tc_head_body,
        grid=(_HEAD // _BB,),
        in_specs=[
            pl.BlockSpec((_G, _HALF), lambda b: (0, 0)),
            pl.BlockSpec((_G, _HALF), lambda b: (0, 0)),
        ],
        out_specs=pl.BlockSpec((_BB, _G, _G, _D), lambda b: (b, 0, 0, 0)),
        out_shape=jax.ShapeDtypeStruct((_BATCH, _G, _G, _D), jnp.float32),
    )(row_table, col_table)

    rowcol = sc_concat(row_table, col_table)

    return pl.pallas_call(
        _tc_tail_body,
        grid=(_TAIL,),
        in_specs=[
            pl.BlockSpec((_G, _D), lambda b: (0, 0)),
            pl.BlockSpec(memory_space=pltpu.MemorySpace.HBM),
        ],
        out_specs=pl.BlockSpec((1, _G, _G, _D), lambda b: (_HEAD + b, 0, 0, 0)),
        out_shape=jax.ShapeDtypeStruct((_BATCH, _G, _G, _D), jnp.float32),
        input_output_aliases={1: 0},
    )(rowcol, part)
